# baseline (speedup 1.0000x reference)
import functools
import math

import jax
import jax.numpy as jnp
from jax import lax
from jax.experimental import pallas as pl
from jax.experimental.pallas import tpu as pltpu
from jax.experimental.pallas import tpu_sc as plsc

_TS = 128
_NB = 6
_PF = 3
_TCB = 2048


def _tc_prep(v, d, d2, scale):
    grid = (v + _TCB - 1) // _TCB

    def body(t_ref, o_ref):
        r = lax.broadcasted_iota(jnp.int32, (d, d), 0)
        c = lax.broadcasted_iota(jnp.int32, (d, d), 1)
        eye_s = jnp.where(r == c, scale, 0.0).astype(jnp.float32)
        o_ref[:, :d] = lax.dot_general(
            t_ref[...], eye_s, (((0,), (0,)), ((), ())),
            preferred_element_type=jnp.float32,
        )

    return pl.pallas_call(
        body,
        grid=(grid,),
        in_specs=[pl.BlockSpec((d, _TCB), lambda i: (0, i))],
        out_specs=pl.BlockSpec((_TCB, d2), lambda i: (i, 0)),
        out_shape=jax.ShapeDtypeStruct((v, d2), jnp.float32),
    )


def _sc_gather(n_tok, d, d2):
    nc, ns = 2, 16
    n_workers = nc * ns
    n_steps = n_tok // n_workers // _TS
    assert _PF < _NB
    mesh = plsc.VectorSubcoreMesh(core_axis_name="c", subcore_axis_name="s")

    @functools.partial(
        pl.kernel,
        mesh=mesh,
        out_type=jax.ShapeDtypeStruct((n_tok, d2), jnp.float32),
        scratch_types=[
            pltpu.VMEM((n_steps, _TS), jnp.int32),
            pltpu.VMEM((_NB, _TS, d2), jnp.float32),
            pltpu.SemaphoreType.DMA((_NB,)),
            pltpu.SemaphoreType.DMA((_NB,)),
        ],
        compiler_params=pltpu.CompilerParams(use_tc_tiling_on_sc=False),
    )
    def emb(idx_hbm, tab_hbm, out_hbm, idx_v, rows, gsem, ssem):
        wid = lax.axis_index("s") * nc + lax.axis_index("c")
        base_step = wid * n_steps
        pltpu.sync_copy(idx_hbm.at[pl.ds(base_step, n_steps)], idx_v)

        def start_gather(step):
            b = step % _NB
            pltpu.async_copy(
                tab_hbm.at[idx_v.at[step]], rows.at[b], gsem.at[b]
            )

        def store_dst(step):
            return out_hbm.at[
                pl.ds((base_step + step) * _TS, _TS), pl.ds(0, d)
            ]

        for s in range(_PF):
            start_gather(s)

        def step_body(s, carry):
            b = s % _NB
            pltpu.make_async_copy(
                tab_hbm.at[idx_v.at[s]], rows.at[b], gsem.at[b]
            ).wait()
            pltpu.async_copy(
                rows.at[b, slice(None), pl.ds(0, d)], store_dst(s),
                ssem.at[b],
            )

            @pl.when(s + _PF >= _NB)
            def _():
                b2 = (s + _PF) % _NB
                pltpu.make_async_copy(
                    rows.at[b2, slice(None), pl.ds(0, d)],
                    store_dst(0),
                    ssem.at[b2],
                ).wait()

            @pl.when(s + _PF < n_steps)
            def _():
                start_gather(s + _PF)

            return carry

        lax.fori_loop(0, n_steps, step_body, 0)

        for s in range(n_steps - (_NB - _PF), n_steps):
            b = s % _NB
            pltpu.make_async_copy(
                rows.at[b, slice(None), pl.ds(0, d)], store_dst(0),
                ssem.at[b],
            ).wait()

    return emb


def kernel(tokens, table):
    n_b, n_l = tokens.shape
    v, d = table.shape
    n = n_b * n_l
    d2 = 2 * d
    scale = math.sqrt(d)
    tab_t = jnp.transpose(table)
    tab_rows = _tc_prep(v, d, d2, scale)(tab_t)
    idx = tokens.reshape(n // _TS, _TS)
    o = _sc_gather(n, d, d2)(idx, tab_rows)
    return o[:, :d].reshape(n_b, n_l, d)


# speedup vs baseline: 1.1140x; 1.1140x over previous
import functools
import math

import jax
import jax.numpy as jnp
from jax import lax
from jax.experimental import pallas as pl
from jax.experimental.pallas import tpu as pltpu
from jax.experimental.pallas import tpu_sc as plsc

_TS = 128
_NB = 8
_PF = 4
_TCB = 2048


def _tc_prep(v, d, d2, scale):
    grid = (v + _TCB - 1) // _TCB

    def body(t_ref, o_ref):
        o_ref[:, :d] = t_ref[...].T * scale

    return pl.pallas_call(
        body,
        grid=(grid,),
        in_specs=[pl.BlockSpec((d, _TCB), lambda i: (0, i))],
        out_specs=pl.BlockSpec((_TCB, d2), lambda i: (i, 0)),
        out_shape=jax.ShapeDtypeStruct((v, d2), jnp.float32),
    )


def _sc_gather(n_tok, d, d2):
    nc, ns = 2, 16
    n_workers = nc * ns
    n_steps = n_tok // n_workers // _TS
    assert _PF < _NB
    mesh = plsc.VectorSubcoreMesh(core_axis_name="c", subcore_axis_name="s")

    @functools.partial(
        pl.kernel,
        mesh=mesh,
        out_type=jax.ShapeDtypeStruct((n_tok, d2), jnp.float32),
        scratch_types=[
            pltpu.VMEM((n_steps, _TS), jnp.int32),
            pltpu.VMEM((_NB, _TS, d), jnp.float32),
            pltpu.SemaphoreType.DMA((_NB,)),
            pltpu.SemaphoreType.DMA((_NB,)),
        ],
        compiler_params=pltpu.CompilerParams(use_tc_tiling_on_sc=False),
    )
    def emb(idx_hbm, tab_hbm, out_hbm, idx_v, rows, gsem, ssem):
        wid = lax.axis_index("s") * nc + lax.axis_index("c")
        base_step = wid * n_steps
        pltpu.sync_copy(idx_hbm.at[pl.ds(base_step, n_steps)], idx_v)

        def dbl(r, carry):
            for c in range(_TS // 16):
                sl = pl.ds(c * 16, 16)
                idx_v[r, sl] = idx_v[r, sl] * 2
            return carry

        lax.fori_loop(0, n_steps, dbl, 0)

        def start_gather(step):
            b = step % _NB
            pltpu.async_copy(
                tab_hbm.at[idx_v.at[step]], rows.at[b], gsem.at[b]
            )

        def store_dst(step):
            return out_hbm.at[
                pl.ds((base_step + step) * _TS, _TS), pl.ds(0, d)
            ]

        for s in range(_PF):
            start_gather(s)

        def step_body(s, carry):
            b = s % _NB
            pltpu.make_async_copy(
                tab_hbm.at[idx_v.at[s]], rows.at[b], gsem.at[b]
            ).wait()
            pltpu.async_copy(rows.at[b], store_dst(s), ssem.at[b])

            @pl.when(s + _PF >= _NB)
            def _():
                b2 = (s + _PF) % _NB
                pltpu.make_async_copy(
                    rows.at[b2], store_dst(0), ssem.at[b2]
                ).wait()

            @pl.when(s + _PF < n_steps)
            def _():
                start_gather(s + _PF)

            return carry

        lax.fori_loop(0, n_steps, step_body, 0)

        for s in range(n_steps - (_NB - _PF), n_steps):
            b = s % _NB
            pltpu.make_async_copy(
                rows.at[b], store_dst(0), ssem.at[b]
            ).wait()

    return emb


def kernel(tokens, table):
    n_b, n_l = tokens.shape
    v, d = table.shape
    n = n_b * n_l
    d2 = 2 * d
    scale = math.sqrt(d)
    tab_t = jnp.transpose(table)
    tab_rows = _tc_prep(v, d, d2, scale)(tab_t)
    idx = tokens.reshape(n // _TS, _TS)
    tab2 = tab_rows.reshape(2 * v, d)
    o = _sc_gather(n, d, d2)(idx, tab2)
    return o[:, :d].reshape(n_b, n_l, d)


# speedup vs baseline: 1.4820x; 1.3304x over previous
import functools
import math

import jax
import jax.numpy as jnp
from jax import lax
from jax.experimental import pallas as pl
from jax.experimental.pallas import tpu as pltpu
from jax.experimental.pallas import tpu_sc as plsc

_TS = 128
_NB = 8
_PF = 4
_TCB = 8192


def _tc_prep(v, d, d2, scale):
    grid = (v + _TCB - 1) // _TCB

    def body(t_ref, o_ref):
        o_ref[:, :d] = t_ref[...].T * scale

    return pl.pallas_call(
        body,
        grid=(grid,),
        in_specs=[pl.BlockSpec((d, _TCB), lambda i: (0, i))],
        out_specs=pl.BlockSpec((_TCB, d2), lambda i: (i, 0)),
        out_shape=jax.ShapeDtypeStruct((v, d2), jnp.float32),
    )


def _sc_gather(n_tok, d, d2):
    nc, ns = 2, 16
    n_workers = nc * ns
    n_steps = n_tok // n_workers // _TS
    assert _PF < _NB
    mesh = plsc.VectorSubcoreMesh(core_axis_name="c", subcore_axis_name="s")

    @functools.partial(
        pl.kernel,
        mesh=mesh,
        out_type=jax.ShapeDtypeStruct((n_tok, d2), jnp.float32),
        scratch_types=[
            pltpu.VMEM((n_steps, _TS), jnp.int32),
            pltpu.VMEM((_NB, _TS, d), jnp.float32),
            pltpu.SemaphoreType.DMA((_NB,)),
            pltpu.SemaphoreType.DMA((_NB,)),
        ],
        compiler_params=pltpu.CompilerParams(use_tc_tiling_on_sc=False),
    )
    def emb(idx_hbm, tab_hbm, out_hbm, idx_v, rows, gsem, ssem):
        wid = lax.axis_index("s") * nc + lax.axis_index("c")
        base_step = wid * n_steps
        pltpu.sync_copy(idx_hbm.at[pl.ds(base_step, n_steps)], idx_v)

        def dbl(r, carry):
            for c in range(_TS // 16):
                sl = pl.ds(c * 16, 16)
                idx_v[r, sl] = idx_v[r, sl] * 2
            return carry

        lax.fori_loop(0, n_steps, dbl, 0)

        def start_gather(step):
            b = step % _NB
            pltpu.async_copy(
                tab_hbm.at[idx_v.at[step]], rows.at[b], gsem.at[b]
            )

        def store_dst(step):
            return out_hbm.at[
                pl.ds((base_step + step) * _TS, _TS), pl.ds(0, d)
            ]

        for s in range(_PF):
            start_gather(s)

        def step_body(s, carry):
            b = s % _NB
            pltpu.make_async_copy(
                tab_hbm.at[idx_v.at[s]], rows.at[b], gsem.at[b]
            ).wait()
            pltpu.async_copy(rows.at[b], store_dst(s), ssem.at[b])

            @pl.when(s + _PF >= _NB)
            def _():
                b2 = (s + _PF) % _NB
                pltpu.make_async_copy(
                    rows.at[b2], store_dst(0), ssem.at[b2]
                ).wait()

            @pl.when(s + _PF < n_steps)
            def _():
                start_gather(s + _PF)

            return carry

        lax.fori_loop(0, n_steps, step_body, 0)

        for s in range(n_steps - (_NB - _PF), n_steps):
            b = s % _NB
            pltpu.make_async_copy(
                rows.at[b], store_dst(0), ssem.at[b]
            ).wait()

    return emb


def kernel(tokens, table):
    n_b, n_l = tokens.shape
    v, d = table.shape
    n = n_b * n_l
    d2 = 2 * d
    scale = math.sqrt(d)
    tab_t = jnp.transpose(table)
    tab_rows = _tc_prep(v, d, d2, scale)(tab_t)
    idx = tokens.reshape(n // _TS, _TS)
    tab2 = tab_rows.reshape(2 * v, d)
    o = _sc_gather(n, d, d2)(idx, tab2)
    return o[:, :d].reshape(n_b, n_l, d)


# speedup vs baseline: 1.5293x; 1.0320x over previous
import functools
import math

import jax
import jax.numpy as jnp
from jax import lax
from jax.experimental import pallas as pl
from jax.experimental.pallas import tpu as pltpu
from jax.experimental.pallas import tpu_sc as plsc

_TS = 128
_NB = 8
_PF = 4
_TCB = 16384


def _tc_prep(v, d, d2, scale):
    grid = (v + _TCB - 1) // _TCB

    def body(t_ref, o_ref):
        o_ref[:, :d] = t_ref[...].T * scale

    return pl.pallas_call(
        body,
        grid=(grid,),
        in_specs=[pl.BlockSpec((d, _TCB), lambda i: (0, i))],
        out_specs=pl.BlockSpec((_TCB, d2), lambda i: (i, 0)),
        out_shape=jax.ShapeDtypeStruct((v, d2), jnp.float32),
    )


def _sc_gather(n_tok, d, d2):
    nc, ns = 2, 16
    n_workers = nc * ns
    n_steps = n_tok // n_workers // _TS
    assert _PF < _NB
    mesh = plsc.VectorSubcoreMesh(core_axis_name="c", subcore_axis_name="s")

    @functools.partial(
        pl.kernel,
        mesh=mesh,
        out_type=jax.ShapeDtypeStruct((n_tok, d2), jnp.float32),
        scratch_types=[
            pltpu.VMEM((n_steps, _TS), jnp.int32),
            pltpu.VMEM((_NB, _TS, d), jnp.float32),
            pltpu.SemaphoreType.DMA((_NB,)),
            pltpu.SemaphoreType.DMA((_NB,)),
        ],
        compiler_params=pltpu.CompilerParams(use_tc_tiling_on_sc=False),
    )
    def emb(idx_hbm, tab_hbm, out_hbm, idx_v, rows, gsem, ssem):
        wid = lax.axis_index("s") * nc + lax.axis_index("c")
        base_step = wid * n_steps
        pltpu.sync_copy(idx_hbm.at[pl.ds(base_step, n_steps)], idx_v)

        def dbl(r, carry):
            for c in range(_TS // 16):
                sl = pl.ds(c * 16, 16)
                idx_v[r, sl] = idx_v[r, sl] * 2
            return carry

        lax.fori_loop(0, n_steps, dbl, 0)

        def start_gather(step):
            b = step % _NB
            pltpu.async_copy(
                tab_hbm.at[idx_v.at[step]], rows.at[b], gsem.at[b]
            )

        def store_dst(step):
            return out_hbm.at[
                pl.ds((base_step + step) * _TS, _TS), pl.ds(0, d)
            ]

        for s in range(_PF):
            start_gather(s)

        def step_body(s, carry):
            b = s % _NB
            pltpu.make_async_copy(
                tab_hbm.at[idx_v.at[s]], rows.at[b], gsem.at[b]
            ).wait()
            pltpu.async_copy(rows.at[b], store_dst(s), ssem.at[b])

            @pl.when(s + _PF >= _NB)
            def _():
                b2 = (s + _PF) % _NB
                pltpu.make_async_copy(
                    rows.at[b2], store_dst(0), ssem.at[b2]
                ).wait()

            @pl.when(s + _PF < n_steps)
            def _():
                start_gather(s + _PF)

            return carry

        lax.fori_loop(0, n_steps, step_body, 0)

        for s in range(n_steps - (_NB - _PF), n_steps):
            b = s % _NB
            pltpu.make_async_copy(
                rows.at[b], store_dst(0), ssem.at[b]
            ).wait()

    return emb


def kernel(tokens, table):
    n_b, n_l = tokens.shape
    v, d = table.shape
    n = n_b * n_l
    d2 = 2 * d
    scale = math.sqrt(d)
    tab_t = jnp.transpose(table)
    tab_rows = _tc_prep(v, d, d2, scale)(tab_t)
    idx = tokens.reshape(n // _TS, _TS)
    tab2 = tab_rows.reshape(2 * v, d)
    o = _sc_gather(n, d, d2)(idx, tab2)
    return o[:, :d].reshape(n_b, n_l, d)


# speedup vs baseline: 1.5468x; 1.0114x over previous
import functools
import math

import jax
import jax.numpy as jnp
from jax import lax
from jax.experimental import pallas as pl
from jax.experimental.pallas import tpu as pltpu
from jax.experimental.pallas import tpu_sc as plsc

_TS = 128
_NB = 8
_PF = 4
_TCB = 32768


def _tc_prep(v, d, d2, scale):
    grid = (v + _TCB - 1) // _TCB

    def body(t_ref, o_ref):
        o_ref[:, :d] = t_ref[...].T * scale

    return pl.pallas_call(
        body,
        grid=(grid,),
        in_specs=[pl.BlockSpec((d, _TCB), lambda i: (0, i))],
        out_specs=pl.BlockSpec((_TCB, d2), lambda i: (i, 0)),
        out_shape=jax.ShapeDtypeStruct((v, d2), jnp.float32),
    )


def _sc_gather(n_tok, d, d2):
    nc, ns = 2, 16
    n_workers = nc * ns
    n_steps = n_tok // n_workers // _TS
    assert _PF < _NB
    mesh = plsc.VectorSubcoreMesh(core_axis_name="c", subcore_axis_name="s")

    @functools.partial(
        pl.kernel,
        mesh=mesh,
        out_type=jax.ShapeDtypeStruct((n_tok, d2), jnp.float32),
        scratch_types=[
            pltpu.VMEM((n_steps, _TS), jnp.int32),
            pltpu.VMEM((_NB, _TS, d), jnp.float32),
            pltpu.SemaphoreType.DMA((_NB,)),
            pltpu.SemaphoreType.DMA((_NB,)),
        ],
        compiler_params=pltpu.CompilerParams(use_tc_tiling_on_sc=False),
    )
    def emb(idx_hbm, tab_hbm, out_hbm, idx_v, rows, gsem, ssem):
        wid = lax.axis_index("s") * nc + lax.axis_index("c")
        base_step = wid * n_steps
        pltpu.sync_copy(idx_hbm.at[pl.ds(base_step, n_steps)], idx_v)

        def dbl(r, carry):
            for c in range(_TS // 16):
                sl = pl.ds(c * 16, 16)
                idx_v[r, sl] = idx_v[r, sl] * 2
            return carry

        lax.fori_loop(0, n_steps, dbl, 0)

        def start_gather(step):
            b = step % _NB
            pltpu.async_copy(
                tab_hbm.at[idx_v.at[step]], rows.at[b], gsem.at[b]
            )

        def store_dst(step):
            return out_hbm.at[
                pl.ds((base_step + step) * _TS, _TS), pl.ds(0, d)
            ]

        for s in range(_PF):
            start_gather(s)

        def step_body(s, carry):
            b = s % _NB
            pltpu.make_async_copy(
                tab_hbm.at[idx_v.at[s]], rows.at[b], gsem.at[b]
            ).wait()
            pltpu.async_copy(rows.at[b], store_dst(s), ssem.at[b])

            @pl.when(s + _PF >= _NB)
            def _():
                b2 = (s + _PF) % _NB
                pltpu.make_async_copy(
                    rows.at[b2], store_dst(0), ssem.at[b2]
                ).wait()

            @pl.when(s + _PF < n_steps)
            def _():
                start_gather(s + _PF)

            return carry

        lax.fori_loop(0, n_steps, step_body, 0)

        for s in range(n_steps - (_NB - _PF), n_steps):
            b = s % _NB
            pltpu.make_async_copy(
                rows.at[b], store_dst(0), ssem.at[b]
            ).wait()

    return emb


def kernel(tokens, table):
    n_b, n_l = tokens.shape
    v, d = table.shape
    n = n_b * n_l
    d2 = 2 * d
    scale = math.sqrt(d)
    tab_t = jnp.transpose(table)
    tab_rows = _tc_prep(v, d, d2, scale)(tab_t)
    idx = tokens.reshape(n // _TS, _TS)
    tab2 = tab_rows.reshape(2 * v, d)
    o = _sc_gather(n, d, d2)(idx, tab2)
    return o[:, :d].reshape(n_b, n_l, d)
